# baseline (speedup 1.0000x reference)
import functools

import jax
import jax.numpy as jnp
from jax.experimental import pallas as pl
from jax.experimental.pallas import tpu as pltpu

B = 64
N = 128
ND = 16
BD = 3
NHID = 128
NOUT = 128
NFLOW = 12
DEQ = 0.9
E = 1458
EPAD = 1536
LP = 8


def _sig_and_logsig(pre):
    u = jnp.exp(-jnp.clip(pre, -80.0, 80.0))
    up1 = 1.0 + u
    ls = jnp.where(pre < -80.0, pre, -jnp.log(up1))
    return 1.0 / up1, ls


def _dotT(a, b):
    return jax.lax.dot_general(a, b, (((0,), (0,)), ((), ())),
                               preferred_element_type=jnp.float32)


def _dotN(a, b):
    return jax.lax.dot_general(a, b, (((1,), (1,)), ((), ())),
                               preferred_element_type=jnp.float32)


def _fused_body(xT, adj_ref, nnT, neT, oct_ref, ort_ref,
                ws0, wr0, ws1, wr1, ws2, wr2,
                nws, nbs, nwt, nbt,
                estop, esbot, ettop, etbot, ebs, ebt,
                zn_ref, ze_ref, ldn_ref, lde_ref):
    f32 = jnp.float32
    dot = functools.partial(jnp.dot, preferred_element_type=f32)

    x = xT[0]
    adj = adj_ref[0]
    OCT = oct_ref[...]
    ORT = ort_ref[...]

    def rgcn(hT, WselfT, WrelT):
        acc = dot(WselfT, hT)
        for r in range(BD):
            inner = _dotN(hT, adj[r])
            acc = acc + dot(WrelT[r], inner)
        return jnp.maximum(acc, 0.0)

    h = rgcn(x, ws0[...], wr0[...])
    h = rgcn(h, ws1[...], wr1[...])
    h = rgcn(h, ws2[...], wr2[...])

    xc = x + DEQ * nnT[0]
    nws_v = nws[...]
    nwt_v = nwt[...]
    nbs_v = nbs[...]
    nbt_v = nbt[...]
    ldn = jnp.zeros((), f32)
    for l in range(NFLOW):
        pre = dot(nws_v[l], h) + nbs_v[l]
        s, ls = _sig_and_logsig(pre)
        t = dot(nwt_v[l], h) + nbt_v[l]
        xc = xc * s + t
        ldn = ldn + jnp.sum(ls)
    zn_ref[0] = xc
    ldn_ref[...] = jnp.zeros((1, 1, 128), f32) + ldn

    acat = jnp.transpose(adj, (1, 0, 2)).reshape(N, BD * N)
    m_all = _dotT(acat, ORT)
    sel = jnp.sum(m_all.reshape(BD, N, EPAD) * OCT[None], axis=1)
    ec = sel + DEQ * neT[0]

    pre_s = dot(dot(estop[...], h), OCT) + dot(dot(esbot[...], h), ORT)
    pre_t = dot(dot(ettop[...], h), OCT) + dot(dot(etbot[...], h), ORT)

    ebs_v = ebs[...]
    ebt_v = ebt[...]
    emask = jax.lax.broadcasted_iota(jnp.int32, (1, EPAD), 1) < E
    lde = jnp.zeros((), f32)
    for l in range(NFLOW):
        ps = pre_s[LP * l:LP * l + LP][:BD] + ebs_v[l]
        s, ls = _sig_and_logsig(ps)
        t = pre_t[LP * l:LP * l + LP][:BD] + ebt_v[l]
        ec = ec * s + t
        lde = lde + jnp.sum(jnp.where(emask, ls, 0.0))
    ze_ref[0] = ec
    lde_ref[...] = jnp.zeros((1, 1, 128), f32) + lde


def kernel(inp_node_features, inp_adj_features, noise_node, noise_edge,
           rgcn_Wself0, rgcn_Wrel0, rgcn_Wself1, rgcn_Wrel1, rgcn_Wself2,
           rgcn_Wrel2, node_Ws, node_bs, node_Wt, node_bt, edge_Ws, edge_bs,
           edge_Wt, edge_bt, rows, cols):
    f32 = jnp.float32

    xT = jnp.transpose(inp_node_features, (0, 2, 1))
    nnT = jnp.transpose(noise_node, (0, 2, 1))
    neP = jnp.pad(noise_edge, ((0, 0), (0, EPAD - E), (0, 0)))
    neT = jnp.transpose(neP, (0, 2, 1))

    rows_p = jnp.pad(rows, (0, EPAD - E), constant_values=-1)
    cols_p = jnp.pad(cols, (0, EPAD - E), constant_values=-1)
    n_iota = jnp.arange(N, dtype=rows_p.dtype)[:, None]
    OCT = (cols_p[None, :] == n_iota).astype(f32)
    ORT = (rows_p[None, :] == n_iota).astype(f32)

    ws0 = rgcn_Wself0.T
    wr0 = jnp.transpose(rgcn_Wrel0, (0, 2, 1))
    ws1 = rgcn_Wself1.T
    wr1 = jnp.transpose(rgcn_Wrel1, (0, 2, 1))
    ws2 = rgcn_Wself2.T
    wr2 = jnp.transpose(rgcn_Wrel2, (0, 2, 1))

    nws = jnp.transpose(node_Ws, (0, 2, 1))
    nwt = jnp.transpose(node_Wt, (0, 2, 1))
    nbs = jnp.broadcast_to(node_bs[:, :, None], (NFLOW, ND, N))
    nbt = jnp.broadcast_to(node_bt[:, :, None], (NFLOW, ND, N))

    def pad_rows(w_half):
        wt = jnp.transpose(w_half, (0, 2, 1))
        return jnp.pad(wt, ((0, 0), (0, LP - BD), (0, 0))).reshape(NFLOW * LP, NOUT)

    estop = pad_rows(edge_Ws[:, :NOUT, :])
    esbot = pad_rows(edge_Ws[:, NOUT:, :])
    ettop = pad_rows(edge_Wt[:, :NOUT, :])
    etbot = pad_rows(edge_Wt[:, NOUT:, :])
    ebs = jnp.broadcast_to(edge_bs[:, :, None], (NFLOW, BD, EPAD))
    ebt = jnp.broadcast_to(edge_bt[:, :, None], (NFLOW, BD, EPAD))

    def rep(shape):
        nd = len(shape)
        return pl.BlockSpec(shape, lambda b, _n=nd: (0,) * _n)

    per_b = lambda shape: pl.BlockSpec((1,) + shape[1:], lambda b: (b,) + (0,) * (len(shape) - 1))

    in_specs = [
        per_b((B, ND, N)),
        per_b((B, BD, N, N)),
        per_b((B, ND, N)),
        per_b((B, BD, EPAD)),
        rep((N, EPAD)),
        rep((N, EPAD)),
        rep((NHID, ND)), rep((BD, NHID, ND)),
        rep((NHID, NHID)), rep((BD, NHID, NHID)),
        rep((NOUT, NHID)), rep((BD, NOUT, NHID)),
        rep((NFLOW, ND, NOUT)), rep((NFLOW, ND, N)),
        rep((NFLOW, ND, NOUT)), rep((NFLOW, ND, N)),
        rep((NFLOW * LP, NOUT)), rep((NFLOW * LP, NOUT)),
        rep((NFLOW * LP, NOUT)), rep((NFLOW * LP, NOUT)),
        rep((NFLOW, BD, EPAD)), rep((NFLOW, BD, EPAD)),
    ]
    out_specs = [
        per_b((B, ND, N)),
        per_b((B, BD, EPAD)),
        pl.BlockSpec((1, 1, 128), lambda b: (b, 0, 0)),
        pl.BlockSpec((1, 1, 128), lambda b: (b, 0, 0)),
    ]
    out_shapes = [
        jax.ShapeDtypeStruct((B, ND, N), f32),
        jax.ShapeDtypeStruct((B, BD, EPAD), f32),
        jax.ShapeDtypeStruct((B, 1, 128), f32),
        jax.ShapeDtypeStruct((B, 1, 128), f32),
    ]

    znT, zeT, ldn, lde = pl.pallas_call(
        _fused_body,
        grid=(B,),
        in_specs=in_specs,
        out_specs=out_specs,
        out_shape=out_shapes,
        compiler_params=pltpu.CompilerParams(
            dimension_semantics=("arbitrary",),
        ),
    )(xT, inp_adj_features, nnT, neT, OCT, ORT,
      ws0, wr0, ws1, wr1, ws2, wr2,
      nws, nbs, nwt, nbt,
      estop, esbot, ettop, etbot, ebs, ebt)

    z_node = jnp.transpose(znT, (0, 2, 1)).reshape(B, N * ND)
    z_edge = jnp.transpose(zeT, (0, 2, 1))[:, :E, :].reshape(B, E * BD)
    return (z_node, z_edge, ldn[:, 0, 0], lde[:, 0, 0])


# speedup vs baseline: 1.0619x; 1.0619x over previous
import functools

import jax
import jax.numpy as jnp
from jax.experimental import pallas as pl
from jax.experimental.pallas import tpu as pltpu
from jax.experimental.pallas import tpu_sc as plsc

B = 64
N = 128
ND = 16
BD = 3
NHID = 128
NOUT = 128
NFLOW = 12
DEQ = 0.9
E = 1458
EPAD = 1536
LP = 8
BIG = 1000.0


NC = 2
NS = 16
NW = NC * NS
PAIRS = B * BD
PPW = PAIRS // NW
CHUNKS = EPAD // 16


def _sel_gather_body(adj_hbm, idx_hbm, sel_hbm, adj_v, idx_v, out_v):
    wid = jax.lax.axis_index("s") * NC + jax.lax.axis_index("c")
    pltpu.sync_copy(idx_hbm, idx_v)
    for p in range(PPW):
        pair = wid * PPW + p
        pltpu.sync_copy(adj_hbm.at[pair], adj_v)
        for i in range(CHUNKS):
            iv = idx_v[pl.ds(i * 16, 16)]
            out_v[pl.ds(i * 16, 16)] = plsc.load_gather(adj_v, [iv])
        pltpu.sync_copy(out_v, sel_hbm.at[pair])


def _sel_gather(adj_flat, idx):
    return pl.kernel(
        _sel_gather_body,
        out_type=jax.ShapeDtypeStruct((PAIRS, EPAD), jnp.float32),
        mesh=plsc.VectorSubcoreMesh(core_axis_name="c", subcore_axis_name="s"),
        compiler_params=pltpu.CompilerParams(needs_layout_passes=False),
        scratch_types=[
            pltpu.VMEM((N * N,), jnp.float32),
            pltpu.VMEM((EPAD,), jnp.int32),
            pltpu.VMEM((EPAD,), jnp.float32),
        ],
    )(adj_flat, idx)


def _sig_and_logsig(pre):
    u = jnp.exp(-jnp.clip(pre, -80.0, 80.0))
    up1 = 1.0 + u
    ls = jnp.where(pre < -80.0, pre, -jnp.log(up1))
    return 1.0 / up1, ls


def _dotN(a, b):
    return jax.lax.dot_general(a, b, (((1,), (1,)), ((), ())),
                               preferred_element_type=jnp.float32)


def _fused_body(xT, adj_ref, selT_ref, nnT, neT, oct_ref, ort_ref,
                ws0, wr0, ws1, wr1, ws2, wr2,
                nws, nbs, nwt, nbt,
                estop, esbot, ettop, etbot, ebs, ebt,
                zn_ref, ze_ref, ldn_ref, lde_ref):
    f32 = jnp.float32
    dot = functools.partial(jnp.dot, preferred_element_type=f32)

    x = xT[0]
    adj = adj_ref[0]
    OCT = oct_ref[...]
    ORT = ort_ref[...]

    def rgcn(hT, WselfT, WrelT):
        acc = dot(WselfT, hT)
        for r in range(BD):
            inner = _dotN(hT, adj[r])
            acc = acc + dot(WrelT[r], inner)
        return jnp.maximum(acc, 0.0)

    h = rgcn(x, ws0[...], wr0[...])
    h = rgcn(h, ws1[...], wr1[...])
    h = rgcn(h, ws2[...], wr2[...])

    PSn = dot(nws[...], h) + nbs[...]
    PTn = dot(nwt[...], h) + nbt[...]
    Sn, LSn = _sig_and_logsig(PSn)
    ldn_ref[...] = jnp.zeros((1, 1, 128), f32) + jnp.sum(LSn)
    xc = x + DEQ * nnT[0]
    for l in range(NFLOW):
        xc = xc * Sn[ND * l:ND * l + ND] + PTn[ND * l:ND * l + ND]
    zn_ref[0] = xc

    ec = selT_ref[0] + DEQ * neT[0]

    lanepen = jnp.where(
        jax.lax.broadcasted_iota(jnp.int32, (1, EPAD), 1) < E, 0.0, 2.0 * BIG)
    PSe = dot(dot(estop[...], h), OCT) + dot(dot(esbot[...], h), ORT) \
        + ebs[...] + lanepen
    PTe = dot(dot(ettop[...], h), OCT) + dot(dot(etbot[...], h), ORT) + ebt[...]
    Se, LSe = _sig_and_logsig(PSe)
    lde_ref[...] = jnp.zeros((1, 1, 128), f32) + jnp.sum(LSe)
    for l in range(NFLOW):
        ec = ec * Se[LP * l:LP * l + LP][:BD] + PTe[LP * l:LP * l + LP][:BD]
    ze_ref[0] = ec


def kernel(inp_node_features, inp_adj_features, noise_node, noise_edge,
           rgcn_Wself0, rgcn_Wrel0, rgcn_Wself1, rgcn_Wrel1, rgcn_Wself2,
           rgcn_Wrel2, node_Ws, node_bs, node_Wt, node_bt, edge_Ws, edge_bs,
           edge_Wt, edge_bt, rows, cols):
    f32 = jnp.float32

    xT = jnp.transpose(inp_node_features, (0, 2, 1))
    nnT = jnp.transpose(noise_node, (0, 2, 1))
    neP = jnp.pad(noise_edge, ((0, 0), (0, EPAD - E), (0, 0)))
    neT = jnp.transpose(neP, (0, 2, 1))

    idx_p = jnp.pad((rows * N + cols).astype(jnp.int32), (0, EPAD - E))
    sel_flat = _sel_gather(inp_adj_features.reshape(PAIRS, N * N), idx_p)
    selT = sel_flat.reshape(B, BD, EPAD)

    rows_p = jnp.pad(rows, (0, EPAD - E), constant_values=-1)
    cols_p = jnp.pad(cols, (0, EPAD - E), constant_values=-1)
    n_iota = jnp.arange(N, dtype=rows_p.dtype)[:, None]
    OCT = (cols_p[None, :] == n_iota).astype(f32)
    ORT = (rows_p[None, :] == n_iota).astype(f32)

    ws0 = rgcn_Wself0.T
    wr0 = jnp.transpose(rgcn_Wrel0, (0, 2, 1))
    ws1 = rgcn_Wself1.T
    wr1 = jnp.transpose(rgcn_Wrel1, (0, 2, 1))
    ws2 = rgcn_Wself2.T
    wr2 = jnp.transpose(rgcn_Wrel2, (0, 2, 1))

    nws = jnp.transpose(node_Ws, (0, 2, 1)).reshape(NFLOW * ND, NOUT)
    nwt = jnp.transpose(node_Wt, (0, 2, 1)).reshape(NFLOW * ND, NOUT)
    nbs = jnp.broadcast_to(node_bs.reshape(NFLOW * ND, 1), (NFLOW * ND, N))
    nbt = jnp.broadcast_to(node_bt.reshape(NFLOW * ND, 1), (NFLOW * ND, N))

    def pad_rows(w_half):
        wt = jnp.transpose(w_half, (0, 2, 1))
        return jnp.pad(wt, ((0, 0), (0, LP - BD), (0, 0))).reshape(NFLOW * LP, NOUT)

    estop = pad_rows(edge_Ws[:, :NOUT, :])
    esbot = pad_rows(edge_Ws[:, NOUT:, :])
    ettop = pad_rows(edge_Wt[:, :NOUT, :])
    etbot = pad_rows(edge_Wt[:, NOUT:, :])
    ebs_p = jnp.pad(edge_bs, ((0, 0), (0, LP - BD)), constant_values=BIG)
    ebt_p = jnp.pad(edge_bt, ((0, 0), (0, LP - BD)))
    ebs = jnp.broadcast_to(ebs_p.reshape(NFLOW * LP, 1), (NFLOW * LP, EPAD))
    ebt = jnp.broadcast_to(ebt_p.reshape(NFLOW * LP, 1), (NFLOW * LP, EPAD))

    def rep(shape):
        nd = len(shape)
        return pl.BlockSpec(shape, lambda b, _n=nd: (0,) * _n)

    per_b = lambda shape: pl.BlockSpec((1,) + shape[1:], lambda b: (b,) + (0,) * (len(shape) - 1))

    in_specs = [
        per_b((B, ND, N)),
        per_b((B, BD, N, N)),
        per_b((B, BD, EPAD)),
        per_b((B, ND, N)),
        per_b((B, BD, EPAD)),
        rep((N, EPAD)),
        rep((N, EPAD)),
        rep((NHID, ND)), rep((BD, NHID, ND)),
        rep((NHID, NHID)), rep((BD, NHID, NHID)),
        rep((NOUT, NHID)), rep((BD, NOUT, NHID)),
        rep((NFLOW * ND, NOUT)), rep((NFLOW * ND, N)),
        rep((NFLOW * ND, NOUT)), rep((NFLOW * ND, N)),
        rep((NFLOW * LP, NOUT)), rep((NFLOW * LP, NOUT)),
        rep((NFLOW * LP, NOUT)), rep((NFLOW * LP, NOUT)),
        rep((NFLOW * LP, EPAD)), rep((NFLOW * LP, EPAD)),
    ]
    out_specs = [
        per_b((B, ND, N)),
        per_b((B, BD, EPAD)),
        pl.BlockSpec((1, 1, 128), lambda b: (b, 0, 0)),
        pl.BlockSpec((1, 1, 128), lambda b: (b, 0, 0)),
    ]
    out_shapes = [
        jax.ShapeDtypeStruct((B, ND, N), f32),
        jax.ShapeDtypeStruct((B, BD, EPAD), f32),
        jax.ShapeDtypeStruct((B, 1, 128), f32),
        jax.ShapeDtypeStruct((B, 1, 128), f32),
    ]

    znT, zeT, ldn, lde = pl.pallas_call(
        _fused_body,
        grid=(B,),
        in_specs=in_specs,
        out_specs=out_specs,
        out_shape=out_shapes,
        compiler_params=pltpu.CompilerParams(
            dimension_semantics=("arbitrary",),
        ),
    )(xT, inp_adj_features, selT, nnT, neT, OCT, ORT,
      ws0, wr0, ws1, wr1, ws2, wr2,
      nws, nbs, nwt, nbt,
      estop, esbot, ettop, etbot, ebs, ebt)

    z_node = jnp.transpose(znT, (0, 2, 1)).reshape(B, N * ND)
    z_edge = jnp.transpose(zeT, (0, 2, 1))[:, :E, :].reshape(B, E * BD)
    return (z_node, z_edge, ldn[:, 0, 0], lde[:, 0, 0])


# speedup vs baseline: 1.9233x; 1.8111x over previous
import functools

import jax
import jax.numpy as jnp
from jax.experimental import pallas as pl
from jax.experimental.pallas import tpu as pltpu
from jax.experimental.pallas import tpu_sc as plsc

B = 64
N = 128
ND = 16
BD = 3
NHID = 128
NOUT = 128
NFLOW = 12
DEQ = 0.9
E = 1458
EPAD = 1536
LP = 4
GPB = 8
BIG = 1000.0


NC = 2
NS = 16
NW = NC * NS
PAIRS = B * BD
PPW = PAIRS // NW
CHUNKS = EPAD // 16


def _sel_gather_body(adj_hbm, idx_hbm, sel_hbm, adj_v, idx_v, out_v):
    wid = jax.lax.axis_index("s") * NC + jax.lax.axis_index("c")
    pltpu.sync_copy(idx_hbm, idx_v)
    for p in range(PPW):
        pair = wid * PPW + p
        pltpu.sync_copy(adj_hbm.at[pair], adj_v)
        for i in range(CHUNKS):
            iv = idx_v[pl.ds(i * 16, 16)]
            out_v[pl.ds(i * 16, 16)] = plsc.load_gather(adj_v, [iv])
        pltpu.sync_copy(out_v, sel_hbm.at[pair])


def _sel_gather(adj_flat, idx):
    return pl.kernel(
        _sel_gather_body,
        out_type=jax.ShapeDtypeStruct((PAIRS, EPAD), jnp.float32),
        mesh=plsc.VectorSubcoreMesh(core_axis_name="c", subcore_axis_name="s"),
        compiler_params=pltpu.CompilerParams(needs_layout_passes=False),
        scratch_types=[
            pltpu.VMEM((N * N,), jnp.float32),
            pltpu.VMEM((EPAD,), jnp.int32),
            pltpu.VMEM((EPAD,), jnp.float32),
        ],
    )(adj_flat, idx)


def _sig_and_logsig(pre):
    u = jnp.exp(-jnp.clip(pre, -80.0, 80.0))
    up1 = 1.0 + u
    ls = jnp.where(pre < -80.0, pre, -jnp.log(up1))
    return 1.0 / up1, ls


def _dotN(a, b):
    return jax.lax.dot_general(a, b, (((1,), (1,)), ((), ())),
                               preferred_element_type=jnp.float32)


def _fused_body(xT, adj_ref, selT_ref, nnT, neT, oct_ref, ort_ref,
                ws0, wr0, ws1, wr1, ws2, wr2,
                nws, nbs, nwt, nbt,
                estop, esbot, ettop, etbot, ebs, ebt,
                zn_ref, ze_ref, ldn_ref, lde_ref):
    f32 = jnp.float32
    dot = functools.partial(jnp.dot, preferred_element_type=f32)

    OCT = oct_ref[...]
    ORT = ort_ref[...]
    lanepen = jnp.where(
        jax.lax.broadcasted_iota(jnp.int32, (1, EPAD), 1) < E, 0.0, 2.0 * BIG)
    cat = functools.partial(jnp.concatenate, axis=1)

    x2 = cat([xT[g] for g in range(GPB)])

    def rgcn(h2, WselfT, WrelT):
        acc = dot(WselfT, h2)
        for r in range(BD):
            inner = cat([_dotN(h2[:, g * N:g * N + N], adj_ref[g, r])
                         for g in range(GPB)])
            acc = acc + dot(WrelT[r], inner)
        return jnp.maximum(acc, 0.0)

    h = rgcn(x2, ws0[...], wr0[...])
    h = rgcn(h, ws1[...], wr1[...])
    h = rgcn(h, ws2[...], wr2[...])

    nbs2 = cat([nbs[...]] * GPB)
    nbt2 = cat([nbt[...]] * GPB)
    PSn = dot(nws[...], h) + nbs2
    PTn = dot(nwt[...], h) + nbt2
    Sn, LSn = _sig_and_logsig(PSn)
    xc = x2 + DEQ * cat([nnT[g] for g in range(GPB)])
    for l in range(NFLOW):
        xc = xc * Sn[ND * l:ND * l + ND] + PTn[ND * l:ND * l + ND]
    for g in range(GPB):
        ldn_ref[g] = jnp.zeros((1, 128), f32) + jnp.sum(LSn[:, g * N:g * N + N])
        zn_ref[g] = xc[:, g * N:g * N + N]

    EsT = dot(estop[...], h)
    EsB = dot(esbot[...], h)
    EtT = dot(ettop[...], h)
    EtB = dot(etbot[...], h)
    for g in range(GPB):
        gs = slice(g * N, g * N + N)
        ec = selT_ref[g] + DEQ * neT[g]
        PSe = dot(EsT[:, gs], OCT) + dot(EsB[:, gs], ORT) + ebs[...] + lanepen
        PTe = dot(EtT[:, gs], OCT) + dot(EtB[:, gs], ORT) + ebt[...]
        Se, LSe = _sig_and_logsig(PSe)
        lde_ref[g] = jnp.zeros((1, 128), f32) + jnp.sum(LSe)
        for l in range(NFLOW):
            ec = ec * Se[LP * l:LP * l + LP][:BD] + PTe[LP * l:LP * l + LP][:BD]
        ze_ref[g] = ec


def kernel(inp_node_features, inp_adj_features, noise_node, noise_edge,
           rgcn_Wself0, rgcn_Wrel0, rgcn_Wself1, rgcn_Wrel1, rgcn_Wself2,
           rgcn_Wrel2, node_Ws, node_bs, node_Wt, node_bt, edge_Ws, edge_bs,
           edge_Wt, edge_bt, rows, cols):
    f32 = jnp.float32

    xT = jnp.transpose(inp_node_features, (0, 2, 1))
    nnT = jnp.transpose(noise_node, (0, 2, 1))
    neP = jnp.pad(noise_edge, ((0, 0), (0, EPAD - E), (0, 0)))
    neT = jnp.transpose(neP, (0, 2, 1))

    idx_p = jnp.pad((rows * N + cols).astype(jnp.int32), (0, EPAD - E))
    sel_flat = _sel_gather(inp_adj_features.reshape(PAIRS, N * N), idx_p)
    selT = sel_flat.reshape(B, BD, EPAD)

    rows_p = jnp.pad(rows, (0, EPAD - E), constant_values=-1)
    cols_p = jnp.pad(cols, (0, EPAD - E), constant_values=-1)
    n_iota = jnp.arange(N, dtype=rows_p.dtype)[:, None]
    OCT = (cols_p[None, :] == n_iota).astype(f32)
    ORT = (rows_p[None, :] == n_iota).astype(f32)

    ws0 = rgcn_Wself0.T
    wr0 = jnp.transpose(rgcn_Wrel0, (0, 2, 1))
    ws1 = rgcn_Wself1.T
    wr1 = jnp.transpose(rgcn_Wrel1, (0, 2, 1))
    ws2 = rgcn_Wself2.T
    wr2 = jnp.transpose(rgcn_Wrel2, (0, 2, 1))

    nws = jnp.transpose(node_Ws, (0, 2, 1)).reshape(NFLOW * ND, NOUT)
    nwt = jnp.transpose(node_Wt, (0, 2, 1)).reshape(NFLOW * ND, NOUT)
    nbs = jnp.broadcast_to(node_bs.reshape(NFLOW * ND, 1), (NFLOW * ND, N))
    nbt = jnp.broadcast_to(node_bt.reshape(NFLOW * ND, 1), (NFLOW * ND, N))

    def pad_rows(w_half):
        wt = jnp.transpose(w_half, (0, 2, 1))
        return jnp.pad(wt, ((0, 0), (0, LP - BD), (0, 0))).reshape(NFLOW * LP, NOUT)

    estop = pad_rows(edge_Ws[:, :NOUT, :])
    esbot = pad_rows(edge_Ws[:, NOUT:, :])
    ettop = pad_rows(edge_Wt[:, :NOUT, :])
    etbot = pad_rows(edge_Wt[:, NOUT:, :])
    ebs_p = jnp.pad(edge_bs, ((0, 0), (0, LP - BD)), constant_values=BIG)
    ebt_p = jnp.pad(edge_bt, ((0, 0), (0, LP - BD)))
    ebs = jnp.broadcast_to(ebs_p.reshape(NFLOW * LP, 1), (NFLOW * LP, EPAD))
    ebt = jnp.broadcast_to(ebt_p.reshape(NFLOW * LP, 1), (NFLOW * LP, EPAD))

    def rep(shape):
        nd = len(shape)
        return pl.BlockSpec(shape, lambda b, _n=nd: (0,) * _n)

    per_b = lambda shape: pl.BlockSpec((GPB,) + shape[1:], lambda b: (b,) + (0,) * (len(shape) - 1))

    in_specs = [
        per_b((B, ND, N)),
        per_b((B, BD, N, N)),
        per_b((B, BD, EPAD)),
        per_b((B, ND, N)),
        per_b((B, BD, EPAD)),
        rep((N, EPAD)),
        rep((N, EPAD)),
        rep((NHID, ND)), rep((BD, NHID, ND)),
        rep((NHID, NHID)), rep((BD, NHID, NHID)),
        rep((NOUT, NHID)), rep((BD, NOUT, NHID)),
        rep((NFLOW * ND, NOUT)), rep((NFLOW * ND, N)),
        rep((NFLOW * ND, NOUT)), rep((NFLOW * ND, N)),
        rep((NFLOW * LP, NOUT)), rep((NFLOW * LP, NOUT)),
        rep((NFLOW * LP, NOUT)), rep((NFLOW * LP, NOUT)),
        rep((NFLOW * LP, EPAD)), rep((NFLOW * LP, EPAD)),
    ]
    out_specs = [
        per_b((B, ND, N)),
        per_b((B, BD, EPAD)),
        pl.BlockSpec((GPB, 1, 128), lambda b: (b, 0, 0)),
        pl.BlockSpec((GPB, 1, 128), lambda b: (b, 0, 0)),
    ]
    out_shapes = [
        jax.ShapeDtypeStruct((B, ND, N), f32),
        jax.ShapeDtypeStruct((B, BD, EPAD), f32),
        jax.ShapeDtypeStruct((B, 1, 128), f32),
        jax.ShapeDtypeStruct((B, 1, 128), f32),
    ]

    znT, zeT, ldn, lde = pl.pallas_call(
        _fused_body,
        grid=(B // GPB,),
        in_specs=in_specs,
        out_specs=out_specs,
        out_shape=out_shapes,
        compiler_params=pltpu.CompilerParams(
            dimension_semantics=("arbitrary",),
        ),
    )(xT, inp_adj_features, selT, nnT, neT, OCT, ORT,
      ws0, wr0, ws1, wr1, ws2, wr2,
      nws, nbs, nwt, nbt,
      estop, esbot, ettop, etbot, ebs, ebt)

    z_node = jnp.transpose(znT, (0, 2, 1)).reshape(B, N * ND)
    z_edge = jnp.transpose(zeT, (0, 2, 1))[:, :E, :].reshape(B, E * BD)
    return (z_node, z_edge, ldn[:, 0, 0], lde[:, 0, 0])


# speedup vs baseline: 2.0587x; 1.0704x over previous
import functools

import jax
import jax.numpy as jnp
from jax.experimental import pallas as pl
from jax.experimental.pallas import tpu as pltpu
from jax.experimental.pallas import tpu_sc as plsc

B = 64
N = 128
ND = 16
BD = 3
NHID = 128
NOUT = 128
NFLOW = 12
DEQ = 0.9
E = 1458
EPAD = 1536
LP = 4
GPB = 8
BIG = 1000.0


NC = 2
NS = 16
NW = NC * NS
PAIRS = B * BD
PPW = PAIRS // NW
CHUNKS = EPAD // 16


def _sel_gather_body(adj_hbm, rows_hbm, cols_hbm, sel_hbm,
                     adj_v, rv_v, cv_v, out_v):
    wid = jax.lax.axis_index("s") * NC + jax.lax.axis_index("c")
    pltpu.sync_copy(rows_hbm, rv_v)
    pltpu.sync_copy(cols_hbm, cv_v)
    for p in range(PPW):
        pair = wid * PPW + p
        pltpu.sync_copy(adj_hbm.at[pair], adj_v)
        for i in range(CHUNKS):
            r16 = rv_v[pl.ds(i * 16, 16)]
            c16 = cv_v[pl.ds(i * 16, 16)]
            out_v[pl.ds(i * 16, 16)] = plsc.load_gather(adj_v, [r16, c16])
        pltpu.sync_copy(out_v, sel_hbm.at[pair])


def _sel_gather(adj_planes, rows_pad, cols_pad):
    return pl.kernel(
        _sel_gather_body,
        out_type=jax.ShapeDtypeStruct((PAIRS, EPAD), jnp.float32),
        mesh=plsc.VectorSubcoreMesh(core_axis_name="c", subcore_axis_name="s"),
        compiler_params=pltpu.CompilerParams(needs_layout_passes=False),
        scratch_types=[
            pltpu.VMEM((N, N), jnp.float32),
            pltpu.VMEM((EPAD,), jnp.int32),
            pltpu.VMEM((EPAD,), jnp.int32),
            pltpu.VMEM((EPAD,), jnp.float32),
        ],
    )(adj_planes, rows_pad, cols_pad)


def _sig_and_logsig(pre):
    u = jnp.exp(-jnp.clip(pre, -80.0, 80.0))
    up1 = 1.0 + u
    ls = jnp.where(pre < -80.0, pre, -jnp.log(up1))
    return 1.0 / up1, ls


def _dotN(a, b):
    return jax.lax.dot_general(a, b, (((1,), (1,)), ((), ())),
                               preferred_element_type=jnp.float32)


def _fused_body(xT, adj_ref, selT_ref, nnT, neT, oct_ref, ort_ref,
                ws0, wr0, ws1, wr1, ws2, wr2,
                nws, nbs, nwt, nbt,
                estop, esbot, ettop, etbot, ebs, ebt,
                zn_ref, ze_ref, ldn_ref, lde_ref):
    f32 = jnp.float32
    dot = functools.partial(jnp.dot, preferred_element_type=f32)
    dote = dot

    OCT = oct_ref[...]
    ORT = ort_ref[...]
    lanepen = jnp.where(
        jax.lax.broadcasted_iota(jnp.int32, (1, EPAD), 1) < E, 0.0, 2.0 * BIG)
    cat = functools.partial(jnp.concatenate, axis=1)

    x2 = cat([xT[g] for g in range(GPB)])

    def rgcn(h2, WselfT, WrelT):
        acc = dot(WselfT, h2)
        for r in range(BD):
            inner = cat([_dotN(h2[:, g * N:g * N + N], adj_ref[g, r])
                         for g in range(GPB)])
            acc = acc + dot(WrelT[r], inner)
        return jnp.maximum(acc, 0.0)

    h = rgcn(x2, ws0[...], wr0[...])
    h = rgcn(h, ws1[...], wr1[...])
    h = rgcn(h, ws2[...], wr2[...])

    nbs2 = cat([nbs[...]] * GPB)
    nbt2 = cat([nbt[...]] * GPB)
    PSn = dote(nws[...], h) + nbs2
    PTn = dote(nwt[...], h) + nbt2
    Sn, LSn = _sig_and_logsig(PSn)
    xc = x2 + DEQ * cat([nnT[g] for g in range(GPB)])
    for l in range(NFLOW):
        xc = xc * Sn[ND * l:ND * l + ND] + PTn[ND * l:ND * l + ND]
    for g in range(GPB):
        ldn_ref[g] = jnp.zeros((1, 128), f32) + jnp.sum(LSn[:, g * N:g * N + N])
        zn_ref[g] = xc[:, g * N:g * N + N]

    EsT = dote(estop[...], h)
    EsB = dote(esbot[...], h)
    EtT = dote(ettop[...], h)
    EtB = dote(etbot[...], h)
    for g in range(GPB):
        gs = slice(g * N, g * N + N)
        ec = selT_ref[g] + DEQ * neT[g]
        PSe = dote(EsT[:, gs], OCT) + dote(EsB[:, gs], ORT) + ebs[...] + lanepen
        PTe = dote(EtT[:, gs], OCT) + dote(EtB[:, gs], ORT) + ebt[...]
        Se, LSe = _sig_and_logsig(PSe)
        lde_ref[g] = jnp.zeros((1, 128), f32) + jnp.sum(LSe)
        for l in range(NFLOW):
            ec = ec * Se[LP * l:LP * l + LP][:BD] + PTe[LP * l:LP * l + LP][:BD]
        ze_ref[g] = ec


def kernel(inp_node_features, inp_adj_features, noise_node, noise_edge,
           rgcn_Wself0, rgcn_Wrel0, rgcn_Wself1, rgcn_Wrel1, rgcn_Wself2,
           rgcn_Wrel2, node_Ws, node_bs, node_Wt, node_bt, edge_Ws, edge_bs,
           edge_Wt, edge_bt, rows, cols):
    f32 = jnp.float32

    xT = jnp.transpose(inp_node_features, (0, 2, 1))
    nnT = jnp.transpose(noise_node, (0, 2, 1))
    neP = jnp.pad(noise_edge, ((0, 0), (0, EPAD - E), (0, 0)))
    neT = jnp.transpose(neP, (0, 2, 1))

    rows_sc = jnp.pad(rows.astype(jnp.int32), (0, EPAD - E))
    cols_sc = jnp.pad(cols.astype(jnp.int32), (0, EPAD - E))
    sel_flat = _sel_gather(inp_adj_features.reshape(PAIRS, N, N),
                           rows_sc, cols_sc)
    selT = sel_flat.reshape(B, BD, EPAD)

    rows_p = jnp.pad(rows, (0, EPAD - E), constant_values=-1)
    cols_p = jnp.pad(cols, (0, EPAD - E), constant_values=-1)
    n_iota = jnp.arange(N, dtype=rows_p.dtype)[:, None]
    OCT = (cols_p[None, :] == n_iota).astype(f32)
    ORT = (rows_p[None, :] == n_iota).astype(f32)

    ws0 = rgcn_Wself0.T
    wr0 = jnp.transpose(rgcn_Wrel0, (0, 2, 1))
    ws1 = rgcn_Wself1.T
    wr1 = jnp.transpose(rgcn_Wrel1, (0, 2, 1))
    ws2 = rgcn_Wself2.T
    wr2 = jnp.transpose(rgcn_Wrel2, (0, 2, 1))

    nws = jnp.transpose(node_Ws, (0, 2, 1)).reshape(NFLOW * ND, NOUT)
    nwt = jnp.transpose(node_Wt, (0, 2, 1)).reshape(NFLOW * ND, NOUT)
    nbs = jnp.broadcast_to(node_bs.reshape(NFLOW * ND, 1), (NFLOW * ND, N))
    nbt = jnp.broadcast_to(node_bt.reshape(NFLOW * ND, 1), (NFLOW * ND, N))

    def pad_rows(w_half):
        wt = jnp.transpose(w_half, (0, 2, 1))
        return jnp.pad(wt, ((0, 0), (0, LP - BD), (0, 0))).reshape(NFLOW * LP, NOUT)

    estop = pad_rows(edge_Ws[:, :NOUT, :])
    esbot = pad_rows(edge_Ws[:, NOUT:, :])
    ettop = pad_rows(edge_Wt[:, :NOUT, :])
    etbot = pad_rows(edge_Wt[:, NOUT:, :])
    ebs_p = jnp.pad(edge_bs, ((0, 0), (0, LP - BD)), constant_values=BIG)
    ebt_p = jnp.pad(edge_bt, ((0, 0), (0, LP - BD)))
    ebs = jnp.broadcast_to(ebs_p.reshape(NFLOW * LP, 1), (NFLOW * LP, EPAD))
    ebt = jnp.broadcast_to(ebt_p.reshape(NFLOW * LP, 1), (NFLOW * LP, EPAD))

    def rep(shape):
        nd = len(shape)
        return pl.BlockSpec(shape, lambda b, _n=nd: (0,) * _n)

    per_b = lambda shape: pl.BlockSpec((GPB,) + shape[1:], lambda b: (b,) + (0,) * (len(shape) - 1))

    in_specs = [
        per_b((B, ND, N)),
        per_b((B, BD, N, N)),
        per_b((B, BD, EPAD)),
        per_b((B, ND, N)),
        per_b((B, BD, EPAD)),
        rep((N, EPAD)),
        rep((N, EPAD)),
        rep((NHID, ND)), rep((BD, NHID, ND)),
        rep((NHID, NHID)), rep((BD, NHID, NHID)),
        rep((NOUT, NHID)), rep((BD, NOUT, NHID)),
        rep((NFLOW * ND, NOUT)), rep((NFLOW * ND, N)),
        rep((NFLOW * ND, NOUT)), rep((NFLOW * ND, N)),
        rep((NFLOW * LP, NOUT)), rep((NFLOW * LP, NOUT)),
        rep((NFLOW * LP, NOUT)), rep((NFLOW * LP, NOUT)),
        rep((NFLOW * LP, EPAD)), rep((NFLOW * LP, EPAD)),
    ]
    out_specs = [
        per_b((B, ND, N)),
        per_b((B, BD, EPAD)),
        pl.BlockSpec((GPB, 1, 128), lambda b: (b, 0, 0)),
        pl.BlockSpec((GPB, 1, 128), lambda b: (b, 0, 0)),
    ]
    out_shapes = [
        jax.ShapeDtypeStruct((B, ND, N), f32),
        jax.ShapeDtypeStruct((B, BD, EPAD), f32),
        jax.ShapeDtypeStruct((B, 1, 128), f32),
        jax.ShapeDtypeStruct((B, 1, 128), f32),
    ]

    znT, zeT, ldn, lde = pl.pallas_call(
        _fused_body,
        grid=(B // GPB,),
        in_specs=in_specs,
        out_specs=out_specs,
        out_shape=out_shapes,
        compiler_params=pltpu.CompilerParams(
            dimension_semantics=("arbitrary",),
        ),
    )(xT, inp_adj_features, selT, nnT, neT, OCT, ORT,
      ws0, wr0, ws1, wr1, ws2, wr2,
      nws, nbs, nwt, nbt,
      estop, esbot, ettop, etbot, ebs, ebt)

    z_node = jnp.transpose(znT, (0, 2, 1)).reshape(B, N * ND)
    z_edge = jnp.transpose(zeT, (0, 2, 1))[:, :E, :].reshape(B, E * BD)
    return (z_node, z_edge, ldn[:, 0, 0], lde[:, 0, 0])
